# baseline (speedup 1.0000x reference)
import jax
import jax.numpy as jnp
from jax import lax
from jax.experimental import pallas as pl
from jax.experimental.pallas import tpu as pltpu
from jax.experimental.pallas import tpu_sc as plsc

B, L, H = 1024, 200, 128
N = B * L
EPS = 1e-12

NC, NS, LANES = 2, 16, 16
NW = NC * NS
NSLICE = 5
SL = N // NSLICE
TPW = SL // NW
CHUNK = 64
NCH = TPW // CHUNK
NBUF = 2
NGRP = CHUNK // LANES

TOK_BLK = 2048
BLKS = SL // TOK_BLK
SEG_V = 16
POS_V = 512


def _sc_body(dids, pids, dtab, ptab_flat, out,
             idxd_v, idxp_v, ptab_v,
             bufd0, bufd1, bufp0, bufp1, bufo0, bufo1,
             semd0, semd1, semo0, semo1):
    bufd = (bufd0, bufd1)
    bufp = (bufp0, bufp1)
    bufo = (bufo0, bufo1)
    semd = (semd0, semd1)
    semo = (semo0, semo1)

    wid = lax.axis_index("s") * NC + lax.axis_index("c")
    pltpu.sync_copy(dids.at[wid], idxd_v)
    pltpu.sync_copy(pids.at[wid], idxp_v)
    pltpu.sync_copy(ptab_flat, ptab_v)

    def start_gather(j, b):
        pltpu.make_async_copy(dtab.at[idxd_v.at[j]], bufd[b], semd[b]).start()

    for b in range(NBUF):
        start_gather(b, b)

    def outer(i, carry):
        j0 = i * NBUF
        for b in range(NBUF):
            j = j0 + b

            for grp in range(NGRP):
                idsv = idxp_v[j, pl.ds(grp * LANES, LANES)] * H
                tokv = (jax.lax.iota(jnp.int32, LANES) + grp * LANES) * H

                def col_body(h, c2):
                    hv = jnp.zeros((LANES,), jnp.int32) + h
                    g = plsc.load_gather(ptab_v, [idsv + hv])
                    plsc.store_scatter(bufp[b], [tokv + hv], g)
                    return c2

                lax.fori_loop(0, H, col_body, 0, unroll=8)

            pltpu.make_async_copy(dtab.at[idxd_v.at[j]], bufd[b], semd[b]).wait()

            @pl.when(i > 0)
            def _():
                pltpu.make_async_copy(
                    bufo[b], out.at[pl.ds(0, CHUNK)], semo[b]).wait()

            def tok_body(t, c2):
                base = t * H
                for h in range(H // LANES):
                    sl = pl.ds(h * LANES, LANES)
                    bufo[b][t, sl] = (bufd[b][t, sl]
                                      + bufp[b][pl.ds(base + h * LANES, LANES)])
                return c2

            lax.fori_loop(0, CHUNK, tok_body, 0, unroll=2)

            @pl.when(j + NBUF < NCH)
            def _():
                start_gather(j + NBUF, b)

            pltpu.make_async_copy(
                bufo[b], out.at[pl.ds(wid * TPW + j * CHUNK, CHUNK)],
                semo[b]).start()
        return carry

    lax.fori_loop(0, NCH // NBUF, outer, 0)

    for b in range(NBUF):
        pltpu.make_async_copy(bufo[b], out.at[pl.ds(0, CHUNK)], semo[b]).wait()


def _sc_gather_sum(dids3, pids3, delays_table, posi_flat):
    mesh = plsc.VectorSubcoreMesh(core_axis_name="c", subcore_axis_name="s")
    f = pl.kernel(
        _sc_body,
        mesh=mesh,
        compiler_params=pltpu.CompilerParams(needs_layout_passes=False),
        out_type=jax.ShapeDtypeStruct((SL, H), jnp.float32),
        scratch_types=[
            pltpu.VMEM((NCH, CHUNK), jnp.int32),
            pltpu.VMEM((NCH, CHUNK), jnp.int32),
            pltpu.VMEM((POS_V * H,), jnp.float32),
            pltpu.VMEM((CHUNK, H), jnp.float32),
            pltpu.VMEM((CHUNK, H), jnp.float32),
            pltpu.VMEM((CHUNK * H,), jnp.float32),
            pltpu.VMEM((CHUNK * H,), jnp.float32),
            pltpu.VMEM((CHUNK, H), jnp.float32),
            pltpu.VMEM((CHUNK, H), jnp.float32),
            pltpu.SemaphoreType.DMA,
            pltpu.SemaphoreType.DMA,
            pltpu.SemaphoreType.DMA,
            pltpu.SemaphoreType.DMA,
        ],
    )
    return f(dids3, pids3, delays_table, posi_flat)


def _ln_math(word, scr, ids, segtab, gamma, beta):
    oh = (ids[:, None] == lax.broadcasted_iota(jnp.int32, (TOK_BLK, SEG_V), 1)
          ).astype(jnp.float32)
    segrows = jnp.dot(oh, segtab, preferred_element_type=jnp.float32)
    s = word + scr + segrows
    mean = jnp.mean(s, axis=-1, keepdims=True)
    c = s - mean
    var = jnp.mean(c * c, axis=-1, keepdims=True)
    return c * jax.lax.rsqrt(var + EPS) * gamma + beta


def _ln_body0(word_ref, scr_ref, seg_ref, segtab_ref, gamma_ref, beta_ref,
              out_ref):
    out_ref[...] = _ln_math(word_ref[...], scr_ref[...], seg_ref[0, 0, :],
                            segtab_ref[...], gamma_ref[...], beta_ref[...])


def _ln_body_acc(acc_ref, word_ref, scr_ref, seg_ref, segtab_ref, gamma_ref,
                 beta_ref, out_ref):
    del acc_ref
    out_ref[...] = _ln_math(word_ref[...], scr_ref[...], seg_ref[0, 0, :],
                            segtab_ref[...], gamma_ref[...], beta_ref[...])


def _tc_ln_slice(s, prev, word_flat, scratch_s, seg3, seg_table, g2, b2):
    common_in = [
        pl.BlockSpec((TOK_BLK, H), lambda i, s=s: (s * BLKS + i, 0)),
        pl.BlockSpec((TOK_BLK, H), lambda i: (i, 0)),
        pl.BlockSpec((1, 1, TOK_BLK), lambda i, s=s: (s * BLKS + i, 0, 0)),
        pl.BlockSpec((SEG_V, H), lambda i: (0, 0)),
        pl.BlockSpec((1, H), lambda i: (0, 0)),
        pl.BlockSpec((1, H), lambda i: (0, 0)),
    ]
    out_spec = pl.BlockSpec((TOK_BLK, H), lambda i, s=s: (s * BLKS + i, 0))
    if prev is None:
        return pl.pallas_call(
            _ln_body0,
            grid=(BLKS,),
            in_specs=common_in,
            out_specs=out_spec,
            out_shape=jax.ShapeDtypeStruct((N, H), jnp.float32),
        )(word_flat, scratch_s, seg3, seg_table, g2, b2)
    return pl.pallas_call(
        _ln_body_acc,
        grid=(BLKS,),
        in_specs=[pl.BlockSpec((8, H), lambda i: (0, 0))] + common_in,
        out_specs=out_spec,
        out_shape=jax.ShapeDtypeStruct((N, H), jnp.float32),
        input_output_aliases={0: 0},
    )(prev, word_flat, scratch_s, seg3, seg_table, g2, b2)


def kernel(word_ids, delays_ids, seg_ids, posi_ids, seg_table, delays_table,
           posi_table, ln_gamma, ln_beta):
    dids4 = delays_ids.reshape(NSLICE, NW, NCH, CHUNK).astype(jnp.int32)
    pids4 = posi_ids.reshape(NSLICE, NW, NCH, CHUNK).astype(jnp.int32)
    seg3 = seg_ids.reshape(N // TOK_BLK, 1, TOK_BLK).astype(jnp.int32)
    word_flat = word_ids.reshape(N, H)
    g2 = ln_gamma.reshape(1, H)
    b2 = ln_beta.reshape(1, H)

    posi_flat = posi_table.reshape(POS_V * H)
    scratches = [
        _sc_gather_sum(dids4[s], pids4[s], delays_table, posi_flat)
        for s in range(NSLICE)
    ]
    out = None
    for s in range(NSLICE):
        out = _tc_ln_slice(s, out, word_flat, scratches[s], seg3, seg_table,
                           g2, b2)
    return out.reshape(B, L, H)


# speedup vs baseline: 1.3716x; 1.3716x over previous
import jax
import jax.numpy as jnp
from jax import lax
from jax.experimental import pallas as pl
from jax.experimental.pallas import tpu as pltpu
from jax.experimental.pallas import tpu_sc as plsc

B, L, H = 1024, 200, 128
N = B * L
EPS = 1e-12

NC, NS, LANES = 2, 16, 16
NW = NC * NS
NSLICE = 1
SL = N // NSLICE
TPW = SL // NW
CHUNK = 32
NCH = TPW // CHUNK
NBUF = 2
HV = H // LANES

SEG_V = 16
POS_V = 512


def _rsqrt16(x):
    i = plsc.bitcast(x, jnp.int32)
    i = jnp.int32(0x5F3759DF) - (i >> 1)
    y = plsc.bitcast(i, jnp.float32)
    for _ in range(3):
        y = y * (1.5 - 0.5 * x * y * y)
    return y


def _sc_body(dids, psids, word, dtab, ptab_flat, stab_flat, gb, out,
             idxd_v, idxps_v, ptab_v, stab_v, gb_v,
             bufd0, bufd1, bufw0, bufw1, bufo0,
             semd0, semd1, semw0, semw1, semo0):
    bufd = (bufd0, bufd1)
    bufw = (bufw0, bufw1)
    bufo = (bufo0, bufo0)
    semd = (semd0, semd1)
    semw = (semw0, semw1)
    semo = (semo0, semo0)

    wid = lax.axis_index("s") * NC + lax.axis_index("c")
    base_tok = wid * TPW
    pltpu.sync_copy(dids.at[wid], idxd_v)
    pltpu.sync_copy(psids.at[wid], idxps_v)
    pltpu.sync_copy(ptab_flat, ptab_v)
    pltpu.sync_copy(stab_flat, stab_v)
    pltpu.sync_copy(gb, gb_v)

    gvec = [gb_v[pl.ds(h * LANES, LANES)] for h in range(HV)]
    bvec = [gb_v[pl.ds(H + h * LANES, LANES)] for h in range(HV)]
    inv_h = jnp.float32(1.0 / H)

    def start_streams(j, b):
        pltpu.make_async_copy(dtab.at[idxd_v.at[j]], bufd[b], semd[b]).start()
        pltpu.make_async_copy(
            word.at[pl.ds(base_tok + j * CHUNK, CHUNK)], bufw[b],
            semw[b]).start()

    for b in range(NBUF):
        start_streams(b, b)

    def outer(i, carry):
        j0 = i * NBUF
        for b in range(NBUF):
            j = j0 + b
            pltpu.make_async_copy(dtab.at[idxd_v.at[j]], bufd[b], semd[b]).wait()
            pltpu.make_async_copy(
                word.at[pl.ds(0, CHUNK)], bufw[b], semw[b]).wait()

            @pl.when(j > 0)
            def _():
                pltpu.make_async_copy(
                    bufo[b], out.at[pl.ds(0, CHUNK)], semo[b]).wait()

            def grp_body(g, c2):
                t0 = g * 2 * LANES
                psvp = idxps_v[j, pl.ds(g * LANES, LANES)]
                pv_even = (psvp & 511) * (H // 2)
                sv_even = ((psvp >> 9) & 15) * H
                pv_odd = ((psvp >> 13) & 511) * (H // 2)
                sv_odd = ((psvp >> 22) & 15) * H
                for tl in range(2 * LANES):
                    t = t0 + tl
                    if tl % 2 == 0:
                        pid = pv_even[tl // 2]
                        sid = sv_even[tl // 2]
                    else:
                        pid = pv_odd[tl // 2]
                        sid = sv_odd[tl // 2]
                    prows = []
                    for hp in range(HV // 2):
                        w = ptab_v[pl.ds(pid + hp * LANES, LANES)]
                        prows.append(plsc.bitcast(w << 16, jnp.float32))
                        prows.append(plsc.bitcast(
                            w & jnp.int32(-65536), jnp.float32))
                    o = []
                    acc_s = jnp.zeros((LANES,), jnp.float32)
                    acc_q = jnp.zeros((LANES,), jnp.float32)
                    for h in range(HV):
                        sl = pl.ds(h * LANES, LANES)
                        v = (bufd[b][t, sl] + bufw[b][t, sl]
                             + prows[h]
                             + stab_v[pl.ds(sid + h * LANES, LANES)])
                        o.append(v)
                        acc_s = acc_s + v
                        acc_q = acc_q + v * v
                    tot = jnp.sum(acc_s, axis=0)
                    tot_q = jnp.sum(acc_q, axis=0)
                    mean = tot * inv_h
                    var = tot_q * inv_h - mean * mean
                    rstd = _rsqrt16(jnp.zeros((LANES,), jnp.float32)
                                    + (var + EPS))
                    for h in range(HV):
                        sl = pl.ds(h * LANES, LANES)
                        bufo[b][t, sl] = ((o[h] - mean) * rstd * gvec[h]
                                          + bvec[h])
                return c2

            lax.fori_loop(0, CHUNK // (2 * LANES), grp_body, 0)

            @pl.when(j + NBUF < NCH)
            def _():
                start_streams(j + NBUF, b)

            pltpu.make_async_copy(
                bufo[b], out.at[pl.ds(base_tok + j * CHUNK, CHUNK)],
                semo[b]).start()
        return carry

    lax.fori_loop(0, NCH // NBUF, outer, 0)

    pltpu.make_async_copy(bufo[0], out.at[pl.ds(0, CHUNK)], semo[0]).wait()


def _sc_fused(dids3, psids3, word_s, delays_table, posi_flat, seg_flat,
              gb_cat):
    mesh = plsc.VectorSubcoreMesh(core_axis_name="c", subcore_axis_name="s")
    f = pl.kernel(
        _sc_body,
        mesh=mesh,
        compiler_params=pltpu.CompilerParams(
            needs_layout_passes=False,
            internal_scratch_in_bytes=256 * 1024),
        out_type=jax.ShapeDtypeStruct((SL, H), jnp.float32),
        scratch_types=[
            pltpu.VMEM((NCH, CHUNK), jnp.int32),
            pltpu.VMEM((NCH, CHUNK // 2), jnp.int32),
            pltpu.VMEM((POS_V * H // 2,), jnp.int32),
            pltpu.VMEM((SEG_V * H,), jnp.float32),
            pltpu.VMEM((2 * H,), jnp.float32),
            pltpu.VMEM((CHUNK, H), jnp.float32),
            pltpu.VMEM((CHUNK, H), jnp.float32),
            pltpu.VMEM((CHUNK, H), jnp.float32),
            pltpu.VMEM((CHUNK, H), jnp.float32),
            pltpu.VMEM((CHUNK, H), jnp.float32),
            pltpu.SemaphoreType.DMA,
            pltpu.SemaphoreType.DMA,
            pltpu.SemaphoreType.DMA,
            pltpu.SemaphoreType.DMA,
            pltpu.SemaphoreType.DMA,
        ],
    )
    return f(dids3, psids3, word_s, delays_table, posi_flat, seg_flat, gb_cat)


def kernel(word_ids, delays_ids, seg_ids, posi_ids, seg_table, delays_table,
           posi_table, ln_gamma, ln_beta):
    dids3 = delays_ids.reshape(NW, NCH, CHUNK).astype(jnp.int32)
    ps = (posi_ids.astype(jnp.int32)
          | (seg_ids.astype(jnp.int32) << 9)).reshape(NW, NCH, CHUNK // 2, 2)
    psids3 = ps[..., 0] | (ps[..., 1] << 13)
    word_flat = word_ids.reshape(N, H)
    pb = posi_table.astype(jnp.bfloat16).reshape(POS_V, H // 32, 2, LANES)
    u16 = jax.lax.bitcast_convert_type(pb, jnp.uint16).astype(jnp.uint32)
    words = u16[:, :, 0, :] | (u16[:, :, 1, :] << 16)
    posi_flat = jax.lax.bitcast_convert_type(
        words.reshape(POS_V * H // 2), jnp.int32)
    seg_flat = seg_table.reshape(SEG_V * H)

    out = _sc_fused(dids3, psids3, word_flat, delays_table, posi_flat,
                    seg_flat, jnp.concatenate([ln_gamma, ln_beta]))
    return out.reshape(B, L, H)
